# baseline (speedup 1.0000x reference)
import jax
import jax.numpy as jnp
from jax.experimental import pallas as pl
from jax.experimental.pallas import tpu as pltpu

_BM = 512
_BK = 1024


def _fused_kernel(b_ref, w_ref, bias_ref, a_ref, out_ref, h_ref):
    i = pl.program_id(0)
    j = pl.program_id(1)

    @pl.when((i == 0) & (j == 0))
    def _():
        h_ref[...] = (
            jnp.dot(b_ref[...], w_ref[...], preferred_element_type=jnp.float32)
            + bias_ref[...]
        )

    partial = jnp.dot(
        a_ref[...],
        h_ref[pl.ds(j * _BK, _BK), :],
        preferred_element_type=jnp.float32,
    )

    @pl.when(j == 0)
    def _():
        out_ref[...] = partial

    @pl.when(j != 0)
    def _():
        out_ref[...] += partial


def kernel(a, b, W, bias):
    n, k = a.shape
    d_in = b.shape[1]
    d_out = W.shape[1]
    bias2d = bias.reshape(1, d_out)

    out = pl.pallas_call(
        _fused_kernel,
        grid=(n // _BM, k // _BK),
        in_specs=[
            pl.BlockSpec((k, d_in), lambda i, j: (0, 0)),
            pl.BlockSpec((d_in, d_out), lambda i, j: (0, 0)),
            pl.BlockSpec((1, d_out), lambda i, j: (0, 0)),
            pl.BlockSpec((_BM, _BK), lambda i, j: (i, j)),
        ],
        out_specs=pl.BlockSpec((_BM, d_out), lambda i, j: (i, 0)),
        out_shape=jax.ShapeDtypeStruct((n, d_out), jnp.float32),
        scratch_shapes=[pltpu.VMEM((k, d_out), jnp.float32)],
        compiler_params=pltpu.CompilerParams(
            dimension_semantics=("arbitrary", "arbitrary"),
        ),
    )(b, W, bias2d, a)
    return out


# speedup vs baseline: 1.2943x; 1.2943x over previous
import jax
import jax.numpy as jnp
from jax.experimental import pallas as pl
from jax.experimental.pallas import tpu as pltpu

_BM = 256


def _fused_kernel(b_ref, w_ref, bias_ref, a_ref, out_ref, h_ref):
    @pl.when(pl.program_id(0) == 0)
    def _():
        h_ref[...] = (
            jnp.dot(b_ref[...], w_ref[...], preferred_element_type=jnp.float32)
            + bias_ref[...]
        )

    out_ref[...] = jnp.dot(
        a_ref[...], h_ref[...], preferred_element_type=jnp.float32
    )


def kernel(a, b, W, bias):
    n, k = a.shape
    d_in = b.shape[1]
    d_out = W.shape[1]
    bias2d = bias.reshape(1, d_out)

    out = pl.pallas_call(
        _fused_kernel,
        grid=(n // _BM,),
        in_specs=[
            pl.BlockSpec((k, d_in), lambda i: (0, 0)),
            pl.BlockSpec((d_in, d_out), lambda i: (0, 0)),
            pl.BlockSpec((1, d_out), lambda i: (0, 0)),
            pl.BlockSpec((_BM, k), lambda i: (i, 0)),
        ],
        out_specs=pl.BlockSpec((_BM, d_out), lambda i: (i, 0)),
        out_shape=jax.ShapeDtypeStruct((n, d_out), jnp.float32),
        scratch_shapes=[pltpu.VMEM((k, d_out), jnp.float32)],
        compiler_params=pltpu.CompilerParams(
            dimension_semantics=("arbitrary",),
        ),
    )(b, W, bias2d, a)
    return out


# speedup vs baseline: 1.3807x; 1.0667x over previous
import jax
import jax.numpy as jnp
from jax.experimental import pallas as pl
from jax.experimental.pallas import tpu as pltpu

_BM = 1024


def _fused_kernel(b_ref, w_ref, bias_ref, a_ref, out_ref, h_ref):
    @pl.when(pl.program_id(0) == 0)
    def _():
        h_ref[...] = (
            jnp.dot(b_ref[...], w_ref[...], preferred_element_type=jnp.float32)
            + bias_ref[...]
        )

    out_ref[...] = jnp.dot(
        a_ref[...], h_ref[...], preferred_element_type=jnp.float32
    )


def kernel(a, b, W, bias):
    n, k = a.shape
    d_in = b.shape[1]
    d_out = W.shape[1]
    bias2d = bias.reshape(1, d_out)

    out = pl.pallas_call(
        _fused_kernel,
        grid=(n // _BM,),
        in_specs=[
            pl.BlockSpec((k, d_in), lambda i: (0, 0)),
            pl.BlockSpec((d_in, d_out), lambda i: (0, 0)),
            pl.BlockSpec((1, d_out), lambda i: (0, 0)),
            pl.BlockSpec((_BM, k), lambda i: (i, 0)),
        ],
        out_specs=pl.BlockSpec((_BM, d_out), lambda i: (i, 0)),
        out_shape=jax.ShapeDtypeStruct((n, d_out), jnp.float32),
        scratch_shapes=[pltpu.VMEM((k, d_out), jnp.float32)],
        compiler_params=pltpu.CompilerParams(
            dimension_semantics=("arbitrary",),
        ),
    )(b, W, bias2d, a)
    return out


# speedup vs baseline: 1.3994x; 1.0135x over previous
import jax
import jax.numpy as jnp
from jax.experimental import pallas as pl
from jax.experimental.pallas import tpu as pltpu

_BM = 512


def _fused_kernel(b_ref, w_ref, bias_ref, a0_ref, a1_ref, out_ref, h_ref):
    @pl.when(pl.program_id(0) == 0)
    def _():
        h_ref[...] = (
            jnp.dot(b_ref[...], w_ref[...], preferred_element_type=jnp.float32)
            + bias_ref[...]
        )

    kh = a0_ref.shape[1]
    out_ref[...] = jnp.dot(
        a0_ref[...], h_ref[:kh, :], preferred_element_type=jnp.float32
    ) + jnp.dot(
        a1_ref[...], h_ref[kh:, :], preferred_element_type=jnp.float32
    )


def kernel(a, b, W, bias):
    n, k = a.shape
    d_in = b.shape[1]
    d_out = W.shape[1]
    kh = k // 2
    bias2d = bias.reshape(1, d_out)

    out = pl.pallas_call(
        _fused_kernel,
        grid=(n // _BM,),
        in_specs=[
            pl.BlockSpec((k, d_in), lambda i: (0, 0)),
            pl.BlockSpec((d_in, d_out), lambda i: (0, 0)),
            pl.BlockSpec((1, d_out), lambda i: (0, 0)),
            pl.BlockSpec((_BM, kh), lambda i: (i, 0)),
            pl.BlockSpec((_BM, kh), lambda i: (i, 1)),
        ],
        out_specs=pl.BlockSpec((_BM, d_out), lambda i: (i, 0)),
        out_shape=jax.ShapeDtypeStruct((n, d_out), jnp.float32),
        scratch_shapes=[pltpu.VMEM((k, d_out), jnp.float32)],
        compiler_params=pltpu.CompilerParams(
            dimension_semantics=("arbitrary",),
        ),
    )(b, W, bias2d, a, a)
    return out
